# baseline (speedup 1.0000x reference)
import jax
import jax.numpy as jnp
from jax import lax
from jax.experimental import pallas as pl
from jax.experimental.pallas import tpu as pltpu
from jax.experimental.pallas import tpu_sc as plsc

BATCH = 16384
FIELDS = 26
NUM_CORES = 2
NUM_SUBCORES = 16
NW = NUM_CORES * NUM_SUBCORES
BPW = BATCH // NW
IPW = BPW * FIELDS
LANES = 16
QUARTERS = 1
BPQ = BPW // QUARTERS
IPQ = BPQ * FIELDS
CHUNKS_Q = BPQ // LANES


def _sc_kernel(xT_hbm, w_hbm, bias_hbm, out_hbm, idx_v, vals_v, out_v,
               bias_v, csem, sem0):
    c = lax.axis_index("c")
    s = lax.axis_index("s")
    wid = s * NUM_CORES + c
    bbase = wid * BPW
    w1 = w_hbm.at[0]
    idx1 = idx_v.at[0]
    sems = (sem0,)

    gathers = []
    for q in range(QUARTERS):
        copies = []
        for f in range(FIELDS):
            copies.append(pltpu.async_copy(
                xT_hbm.at[pl.ds(f, 1), pl.ds(bbase + q * BPQ, BPQ)],
                idx_v.at[pl.ds(0, 1), pl.ds(q * IPQ + f * BPQ, BPQ)],
                csem))
        for cp in copies:
            cp.wait()
        gathers.append(
            pltpu.async_copy(w1.at[idx1.at[pl.ds(q * IPQ, IPQ)]],
                             vals_v.at[pl.ds(q * IPQ, IPQ)], sems[q]))
    pltpu.sync_copy(bias_hbm, bias_v)
    bvec = bias_v[...]

    for q in range(QUARTERS):
        gathers[q].wait()

        def chunk_body(ci, _, q=q):
            off = ci * LANES
            half = FIELDS // 2
            acc0 = bvec + vals_v[pl.ds(q * IPQ + half * BPQ + off, LANES)]
            acc1 = vals_v[pl.ds(q * IPQ + (half + 1) * BPQ + off, LANES)]
            acc2 = vals_v[pl.ds(q * IPQ + (half + 2) * BPQ + off, LANES)]
            for f in range(half):
                acc0 = acc0 + vals_v[pl.ds(q * IPQ + f * BPQ + off, LANES)]
                if half + 3 + f < FIELDS:
                    acc1 = acc1 + vals_v[
                        pl.ds(q * IPQ + (half + 3 + f) * BPQ + off, LANES)]
            out_v[pl.ds(q * BPQ + off, LANES)] = acc0 + acc1 + acc2
            return 0

        lax.fori_loop(0, CHUNKS_Q, chunk_body, 0, unroll=False)

    pltpu.sync_copy(out_v, out_hbm.at[pl.ds(bbase, BPW)])


@jax.jit
def kernel(x, W, bias):
    xT = x.T
    wT = W.T
    bias16 = jnp.broadcast_to(bias, (LANES,))

    mesh = plsc.VectorSubcoreMesh(core_axis_name="c", subcore_axis_name="s")
    run = pl.kernel(
        _sc_kernel,
        mesh=mesh,
        out_type=jax.ShapeDtypeStruct((BATCH,), jnp.float32),
        scratch_types=[
            pltpu.VMEM((1, IPW), jnp.int32),
            pltpu.VMEM((IPW,), jnp.float32),
            pltpu.VMEM((BPW,), jnp.float32),
            pltpu.VMEM((LANES,), jnp.float32),
            pltpu.SemaphoreType.DMA,
            pltpu.SemaphoreType.DMA,
        ],
    )
    return run(xT, wT, bias16).reshape(BATCH, 1)


# speedup vs baseline: 1.0009x; 1.0009x over previous
import jax
import jax.numpy as jnp
from jax import lax
from jax.experimental import pallas as pl
from jax.experimental.pallas import tpu as pltpu
from jax.experimental.pallas import tpu_sc as plsc

BATCH = 16384
FIELDS = 26
NUM_CORES = 2
NUM_SUBCORES = 16
NW = NUM_CORES * NUM_SUBCORES
BPW = BATCH // NW
IPW = BPW * FIELDS
LANES = 16
CHUNKS = BPW // LANES


def _sc_kernel(xT_hbm, w_hbm, bias_hbm, out_hbm, idx_v, vals_v, out_v,
               bias_v, csem, gsem):
    c = lax.axis_index("c")
    s = lax.axis_index("s")
    wid = s * NUM_CORES + c
    bbase = wid * BPW
    w1 = w_hbm.at[0]
    idx1 = idx_v.at[0]

    copies = [
        pltpu.async_copy(
            xT_hbm.at[pl.ds(f, 1), pl.ds(bbase, BPW)],
            idx_v.at[pl.ds(0, 1), pl.ds(f * BPW, BPW)],
            csem)
        for f in range(FIELDS)
    ]
    for cp in copies:
        cp.wait()
    gather = pltpu.async_copy(w1.at[idx1], vals_v, gsem)
    pltpu.sync_copy(bias_hbm, bias_v)
    bvec = bias_v[...]
    gather.wait()

    def chunk_body(ci, _):
        off = ci * LANES
        half = FIELDS // 2
        acc0 = bvec + vals_v[pl.ds(half * BPW + off, LANES)]
        acc1 = vals_v[pl.ds((half + 1) * BPW + off, LANES)]
        acc2 = vals_v[pl.ds((half + 2) * BPW + off, LANES)]
        for f in range(half):
            acc0 = acc0 + vals_v[pl.ds(f * BPW + off, LANES)]
            if half + 3 + f < FIELDS:
                acc1 = acc1 + vals_v[pl.ds((half + 3 + f) * BPW + off, LANES)]
        out_v[pl.ds(off, LANES)] = acc0 + acc1 + acc2
        return 0

    lax.fori_loop(0, CHUNKS, chunk_body, 0, unroll=False)

    pltpu.sync_copy(out_v, out_hbm.at[pl.ds(bbase, BPW)])


@jax.jit
def kernel(x, W, bias):
    xT = x.T
    wT = W.T
    bias16 = jnp.broadcast_to(bias, (LANES,))

    mesh = plsc.VectorSubcoreMesh(core_axis_name="c", subcore_axis_name="s")
    run = pl.kernel(
        _sc_kernel,
        mesh=mesh,
        out_type=jax.ShapeDtypeStruct((BATCH,), jnp.float32),
        scratch_types=[
            pltpu.VMEM((1, IPW), jnp.int32),
            pltpu.VMEM((IPW,), jnp.float32),
            pltpu.VMEM((BPW,), jnp.float32),
            pltpu.VMEM((LANES,), jnp.float32),
            pltpu.SemaphoreType.DMA,
            pltpu.SemaphoreType.DMA,
        ],
    )
    return run(xT, wT, bias16).reshape(BATCH, 1)
